# baseline (speedup 1.0000x reference)
import jax
import jax.numpy as jnp
from jax.experimental import pallas as pl

_VOCAB = 1000
_ROWS = 4096 * 20
_BR = 512
_NB = _ROWS // _BR


def _onehot_body(x_ref, o_ref):
    idx = x_ref[...]
    cols = jax.lax.broadcasted_iota(jnp.int32, (_BR, _VOCAB), 1)
    o_ref[...] = (cols == idx).astype(jnp.float32)


def kernel(x):
    xf = x.reshape(_ROWS, 1).astype(jnp.int32)
    out = pl.pallas_call(
        _onehot_body,
        grid=(_NB,),
        in_specs=[pl.BlockSpec((_BR, 1), lambda i: (i, 0))],
        out_specs=pl.BlockSpec((_BR, _VOCAB), lambda i: (i, 0)),
        out_shape=jax.ShapeDtypeStruct((_ROWS, _VOCAB), jnp.float32),
    )(xf)
    return out.reshape(4096, 20, _VOCAB)


# speedup vs baseline: 1.7048x; 1.7048x over previous
import jax
import jax.numpy as jnp
from jax.experimental import pallas as pl

_VOCAB = 1000
_N = 4096
_K = 20
_B0 = 32
_NB = _N // _B0


def _onehot_body(x_ref, o_ref):
    idx = x_ref[...]
    cols = jax.lax.broadcasted_iota(jnp.int32, (_B0, _K, _VOCAB), 2)
    o_ref[...] = (cols == idx[:, :, None]).astype(jnp.float32)


def kernel(x):
    return pl.pallas_call(
        _onehot_body,
        grid=(_NB,),
        in_specs=[pl.BlockSpec((_B0, _K), lambda i: (i, 0))],
        out_specs=pl.BlockSpec((_B0, _K, _VOCAB), lambda i: (i, 0, 0)),
        out_shape=jax.ShapeDtypeStruct((_N, _K, _VOCAB), jnp.float32),
    )(x.astype(jnp.int32))


# speedup vs baseline: 1.7493x; 1.0261x over previous
import jax
import jax.numpy as jnp
from jax.experimental import pallas as pl
from jax.experimental.pallas import tpu as pltpu

_VOCAB = 1000
_N = 4096
_K = 20
_B0 = 32
_NB = _N // _B0
_NBUF = 4


def _onehot_body(x_ref, o_hbm, buf, sems):
    i = pl.program_id(0)
    slot = jax.lax.rem(i, _NBUF)

    @pl.when(i >= _NBUF)
    def _wait_prev():
        j = i - _NBUF
        pltpu.make_async_copy(
            buf.at[slot], o_hbm.at[pl.ds(j * _B0, _B0)], sems.at[slot]
        ).wait()

    idx = x_ref[...]
    cols = jax.lax.broadcasted_iota(jnp.int32, (_B0, _K, _VOCAB), 2)
    buf[slot] = (cols == idx[:, :, None]).astype(jnp.float32)

    pltpu.make_async_copy(
        buf.at[slot], o_hbm.at[pl.ds(i * _B0, _B0)], sems.at[slot]
    ).start()

    @pl.when(i == _NB - 1)
    def _drain():
        for k in range(_NBUF):
            j = _NB - _NBUF + k
            pltpu.make_async_copy(
                buf.at[k], o_hbm.at[pl.ds(j * _B0, _B0)], sems.at[k]
            ).wait()


def kernel(x):
    return pl.pallas_call(
        _onehot_body,
        grid=(_NB,),
        in_specs=[pl.BlockSpec((_B0, _K), lambda i: (i, 0))],
        out_specs=pl.BlockSpec(memory_space=pltpu.MemorySpace.HBM),
        out_shape=jax.ShapeDtypeStruct((_N, _K, _VOCAB), jnp.float32),
        scratch_shapes=[
            pltpu.VMEM((_NBUF, _B0, _K, _VOCAB), jnp.float32),
            pltpu.SemaphoreType.DMA((_NBUF,)),
        ],
    )(x.astype(jnp.int32))
